# baseline (speedup 1.0000x reference)
import functools

import jax
import jax.numpy as jnp
from jax import lax
from jax.experimental import pallas as pl
from jax.experimental.pallas import tpu as pltpu
from jax.experimental.pallas import tpu_sc as plsc

_LANES = 16
_CHUNK = 32768


def _sc_scale_body(n_chunks, scale, emb_hbm, out_hbm, buf):
    nc = 2
    wid = lax.axis_index("s") * nc + lax.axis_index("c")
    per_w = n_chunks * _CHUNK
    base = wid * per_w
    for c in range(n_chunks):
        off = base + c * _CHUNK
        pltpu.sync_copy(emb_hbm.at[pl.ds(off, _CHUNK)], buf)

        def body(j, _):
            sl = pl.ds(j * _LANES, _LANES)
            buf[sl] = buf[sl] * scale
            return 0

        lax.fori_loop(0, _CHUNK // _LANES, body, 0, unroll=8)
        pltpu.sync_copy(buf, out_hbm.at[pl.ds(off, _CHUNK)])


def kernel(x, emb):
    seq_len = x.shape[1]
    dim = emb.shape[1]
    scale = dim ** (-0.5)
    n = seq_len * dim
    n_workers = 32
    n_chunks = n // (n_workers * _CHUNK)
    emb_flat = emb[:seq_len].reshape(n)

    mesh = plsc.VectorSubcoreMesh(core_axis_name="c", subcore_axis_name="s")
    sc_call = pl.kernel(
        functools.partial(_sc_scale_body, n_chunks, scale),
        mesh=mesh,
        out_type=jax.ShapeDtypeStruct((n,), emb.dtype),
        scratch_types=[pltpu.VMEM((_CHUNK,), jnp.float32)],
    )
    return sc_call(emb_flat).reshape(seq_len, dim)
